# baseline (speedup 1.0000x reference)
import functools
import jax
import jax.numpy as jnp
from jax import lax
from jax.experimental import pallas as pl
from jax.experimental.pallas import tpu as pltpu
from jax.experimental.pallas import tpu_sc as plsc

VOCAB = 1000000
DIM = 32
BATCH = 16384
HIST = 50

NC = 2
NS = 16
NW = NC * NS

PAD_ROWS = 4 * VOCAB
B_PER_W = BATCH // NW
GB = 8
GROUPS = B_PER_W // GB


def _make_gather():
    mesh = plsc.VectorSubcoreMesh(core_axis_name="c", subcore_axis_name="s")

    @functools.partial(
        pl.kernel,
        mesh=mesh,
        out_type=jax.ShapeDtypeStruct((BATCH, HIST, DIM), jnp.float32),
        scratch_types=[
            pltpu.VMEM((B_PER_W, HIST), jnp.int32),
            pltpu.VMEM((2, GB, HIST, DIM), jnp.float32),
            pltpu.SemaphoreType.DMA,
            pltpu.SemaphoreType.DMA,
        ],
        compiler_params=pltpu.CompilerParams(use_tc_tiling_on_sc=False),
    )
    def gather1(idx_hbm, tab_hbm, out, idx_v, buf, sem_g, sem_o):
        wid = lax.axis_index("s") * NC + lax.axis_index("c")
        b0 = wid * B_PER_W

        pltpu.sync_copy(idx_hbm.at[pl.ds(b0, B_PER_W)], idx_v)

        def fire_gathers(grp, slot):
            for j in range(GB):
                b = grp * GB + j
                pltpu.async_copy(tab_hbm.at[idx_v.at[b]],
                                 buf.at[slot].at[j], sem_g)

        def drain_gathers():
            for j in range(GB):
                pltpu.make_async_copy(tab_hbm.at[idx_v.at[0]],
                                      buf.at[0].at[j], sem_g).wait()

        def fire_writes(grp, slot):
            d = pl.ds(b0 + grp * GB, GB)
            pltpu.async_copy(buf.at[slot], out.at[d], sem_o)

        def drain_writes():
            pltpu.make_async_copy(buf.at[0], out.at[pl.ds(0, GB)],
                                  sem_o).wait()

        fire_gathers(0, 0)

        def body(g, carry):
            slot = lax.rem(g, 2)
            nslot = lax.rem(g + 1, 2)

            @pl.when(g >= 1)
            def _():
                drain_writes()

            @pl.when(g + 1 < GROUPS)
            def _():
                fire_gathers(g + 1, nslot)

            drain_gathers()
            fire_writes(g, slot)
            return carry

        lax.fori_loop(0, GROUPS, body, 0)
        drain_writes()

    return gather1


_gather1 = _make_gather()


@jax.jit
def kernel(text, W_word, W_ctx):
    idx4 = (text.astype(jnp.int32) * 4)
    Wp = jnp.pad(W_word, ((0, 0), (0, 96))).reshape(PAD_ROWS, DIM)
    Cp = jnp.pad(W_ctx, ((0, 0), (0, 96))).reshape(PAD_ROWS, DIM)
    contextMatrix = _gather1(idx4, Cp)
    wordMatrix = _gather1(idx4, Wp)
    return (contextMatrix, wordMatrix)


# speedup vs baseline: 1.2956x; 1.2956x over previous
import functools
import jax
import jax.numpy as jnp
from jax import lax
from jax.experimental import pallas as pl
from jax.experimental.pallas import tpu as pltpu
from jax.experimental.pallas import tpu_sc as plsc

VOCAB = 1000000
DIM = 32
BATCH = 16384
HIST = 50

NC = 2
NS = 16
NW = NC * NS

B_PER_W = BATCH // NW
GB = 8
GROUPS = B_PER_W // GB

TBLK = 8192
QBLK = TBLK // 4
NBLK = -(-VOCAB // TBLK)
PROWS = NBLK * TBLK


def _relayout(wt):
    def kern(x_ref, y_ref):
        x = x_ref[...]
        y_ref[...] = jnp.concatenate(
            [x[:, q * QBLK:(q + 1) * QBLK].T for q in range(4)], axis=1)

    return pl.pallas_call(
        kern,
        grid=(NBLK,),
        in_specs=[pl.BlockSpec((DIM, TBLK), lambda i: (0, i))],
        out_specs=pl.BlockSpec((QBLK, 4 * DIM), lambda i: (i, 0)),
        out_shape=jax.ShapeDtypeStruct((PROWS // 4, 4 * DIM), jnp.float32),
    )(wt)


def _make_gather():
    mesh = plsc.VectorSubcoreMesh(core_axis_name="c", subcore_axis_name="s")

    @functools.partial(
        pl.kernel,
        mesh=mesh,
        out_type=jax.ShapeDtypeStruct((BATCH, HIST, DIM), jnp.float32),
        scratch_types=[
            pltpu.VMEM((B_PER_W, HIST), jnp.int32),
            pltpu.VMEM((2, GB, HIST, DIM), jnp.float32),
            pltpu.SemaphoreType.DMA,
            pltpu.SemaphoreType.DMA,
        ],
        compiler_params=pltpu.CompilerParams(use_tc_tiling_on_sc=False),
    )
    def gather1(idx_hbm, tab_hbm, out, idx_v, buf, sem_g, sem_o):
        wid = lax.axis_index("s") * NC + lax.axis_index("c")
        b0 = wid * B_PER_W

        pltpu.sync_copy(idx_hbm.at[pl.ds(b0, B_PER_W)], idx_v)

        def fire_gathers(grp, slot):
            for j in range(GB):
                b = grp * GB + j
                pltpu.async_copy(tab_hbm.at[idx_v.at[b]],
                                 buf.at[slot].at[j], sem_g)

        def drain_gathers():
            for j in range(GB):
                pltpu.make_async_copy(tab_hbm.at[idx_v.at[0]],
                                      buf.at[0].at[j], sem_g).wait()

        def fire_writes(grp, slot):
            d = pl.ds(b0 + grp * GB, GB)
            pltpu.async_copy(buf.at[slot], out.at[d], sem_o)

        def drain_writes():
            pltpu.make_async_copy(buf.at[0], out.at[pl.ds(0, GB)],
                                  sem_o).wait()

        fire_gathers(0, 0)

        def body(g, carry):
            slot = lax.rem(g, 2)
            nslot = lax.rem(g + 1, 2)

            @pl.when(g >= 1)
            def _():
                drain_writes()

            @pl.when(g + 1 < GROUPS)
            def _():
                fire_gathers(g + 1, nslot)

            drain_gathers()
            fire_writes(g, slot)
            return carry

        lax.fori_loop(0, GROUPS, body, 0)
        drain_writes()

    return gather1


_gather1 = _make_gather()


@jax.jit
def kernel(text, W_word, W_ctx):
    v = text.astype(jnp.int32)
    idx = (v & ~(TBLK - 1)) + 4 * (v % QBLK) + (v // QBLK) % 4
    Cp = _relayout(W_ctx.T).reshape(PROWS, DIM)
    Wp = _relayout(W_word.T).reshape(PROWS, DIM)
    contextMatrix = _gather1(idx, Cp)
    wordMatrix = _gather1(idx, Wp)
    return (contextMatrix, wordMatrix)
